# baseline (speedup 1.0000x reference)
import functools

import jax
import jax.numpy as jnp
from jax import lax
from jax.experimental import pallas as pl
from jax.experimental.pallas import tpu as pltpu
from jax.experimental.pallas import tpu_sc as plsc

N_ATOMS = 100000
D = 128
NC = 2
NS = 16
NW = NC * NS
CHUNK = 128
PER_W = 3200
NSTEP = PER_W // CHUNK
NBUF = 6


def _sc_gather(table, idx):
  mesh = plsc.VectorSubcoreMesh(core_axis_name="c", subcore_axis_name="s")

  @functools.partial(
      pl.kernel,
      mesh=mesh,
      out_type=jax.ShapeDtypeStruct((N_ATOMS, D), jnp.float32),
      scratch_types=(
          [pltpu.VMEM_SHARED((93, D), jnp.float32),
           pltpu.VMEM((PER_W,), jnp.int32),
           pltpu.VMEM((NBUF * CHUNK, D), jnp.float32),
           pltpu.SemaphoreType.DMA]
          + [pltpu.SemaphoreType.DMA for _ in range(NBUF)]
          + [pltpu.SemaphoreType.DMA for _ in range(2)]
      ),
  )
  def k(table_hbm, idx_hbm, out_hbm, table_sp, idx_v, ring, isem, *sems):
    gsems = sems[:NBUF]
    ssems = sems[NBUF:]

    wid = lax.axis_index("s") * NC + lax.axis_index("c")
    base = jnp.minimum(wid * PER_W, N_ATOMS - PER_W)

    @pl.when(lax.axis_index("s") == 0)
    def _():
      pltpu.sync_copy(table_hbm, table_sp)

    pltpu.async_copy(idx_hbm.at[pl.ds(base, PER_W)], idx_v, isem).wait()
    for v in range(PER_W // 16):
      idx_v[pl.ds(v * 16, 16)] = idx_v[pl.ds(v * 16, 16)] - 1
    plsc.subcore_barrier()

    NPAIR = NSTEP // 2
    gh = [None] * NSTEP
    sh = [None] * (NPAIR + 1)

    HBM_CHUNKS = frozenset((5, 11, 17, 23))

    def gather(i):
      s = i % NBUF
      src = table_hbm if i in HBM_CHUNKS else table_sp
      return pltpu.async_copy(
          src.at[idx_v.at[pl.ds(i * CHUNK, CHUNK)]],
          ring.at[pl.ds(s * CHUNK, CHUNK)], gsems[s])

    def writeback(p, nchunks):
      s = (2 * p) % NBUF
      return pltpu.async_copy(
          ring.at[pl.ds(s * CHUNK, nchunks * CHUNK)],
          out_hbm.at[pl.ds(base + 2 * p * CHUNK, nchunks * CHUNK)],
          ssems[p % 2])

    for i in range(NSTEP):
      if i >= NBUF and i % 2 == 0:
        sh[(i - NBUF) // 2].wait()
      gh[i] = gather(i)
      if i % 2 == 1 and i >= 3:
        p = (i - 3) // 2
        gh[2 * p].wait()
        gh[2 * p + 1].wait()
        sh[p] = writeback(p, 2)
    gh[NSTEP - 3].wait()
    gh[NSTEP - 2].wait()
    sh[NPAIR - 1] = writeback(NPAIR - 1, 2)
    gh[NSTEP - 1].wait()
    sh[NPAIR] = writeback(NPAIR, 1)
    sh[NPAIR - 2].wait()
    sh[NPAIR - 1].wait()
    sh[NPAIR].wait()

  return k(table, idx)


def kernel(inputs, embeddings):
  return _sc_gather(embeddings.astype(jnp.float32), inputs.astype(jnp.int32))


# speedup vs baseline: 1.6321x; 1.6321x over previous
import functools

import jax
import jax.numpy as jnp
from jax import lax
from jax.experimental import pallas as pl
from jax.experimental.pallas import tpu as pltpu
from jax.experimental.pallas import tpu_sc as plsc

N_ATOMS = 100000
D = 128
NC = 2
NS = 16
NW = NC * NS
CHUNK = 128
PER_W = 3200
NSTEP = PER_W // CHUNK
NBUF = 6


def _sc_gather(table, idx):
  mesh = plsc.VectorSubcoreMesh(core_axis_name="c", subcore_axis_name="s")

  @functools.partial(
      pl.kernel,
      mesh=mesh,
      out_type=jax.ShapeDtypeStruct((N_ATOMS, D), jnp.float32),
      scratch_types=(
          [pltpu.VMEM_SHARED((93, D), jnp.float32),
           pltpu.VMEM((PER_W,), jnp.int32),
           pltpu.VMEM((NBUF * CHUNK, D), jnp.float32),
           pltpu.SemaphoreType.DMA]
          + [pltpu.SemaphoreType.DMA for _ in range(NBUF)]
          + [pltpu.SemaphoreType.DMA for _ in range(2)]
      ),
  )
  def k(table_hbm, idx_hbm, out_hbm, table_sp, idx_v, ring, isem, *sems):
    gsems = sems[:NBUF]
    ssems = sems[NBUF:]

    wid = lax.axis_index("s") * NC + lax.axis_index("c")
    base = jnp.minimum(wid * PER_W, N_ATOMS - PER_W)

    @pl.when(lax.axis_index("s") == 0)
    def _():
      pltpu.sync_copy(table_hbm, table_sp)

    pltpu.async_copy(idx_hbm.at[pl.ds(base, PER_W)], idx_v, isem).wait()
    for v in range(PER_W // 16):
      idx_v[pl.ds(v * 16, 16)] = idx_v[pl.ds(v * 16, 16)] - 1
    plsc.subcore_barrier()

    NBATCH = NSTEP // 3
    gh = [None] * NSTEP
    sh = [None] * (NBATCH + 1)

    def gather(i):
      s = i % NBUF
      return pltpu.async_copy(
          table_sp.at[idx_v.at[pl.ds(i * CHUNK, CHUNK)]],
          ring.at[pl.ds(s * CHUNK, CHUNK)], gsems[s])

    def writeback(b, nchunks):
      s = (3 * b) % NBUF
      return pltpu.async_copy(
          ring.at[pl.ds(s * CHUNK, nchunks * CHUNK)],
          out_hbm.at[pl.ds(base + 3 * b * CHUNK, nchunks * CHUNK)],
          ssems[b % 2])

    for i in range(NSTEP):
      if i >= NBUF and (i - NBUF) % 3 == 0:
        sh[(i - NBUF) // 3].wait()
      gh[i] = gather(i)
      if i % 3 == 2 and i >= 5:
        b = (i - 5) // 3
        gh[3 * b].wait()
        gh[3 * b + 1].wait()
        gh[3 * b + 2].wait()
        sh[b] = writeback(b, 3)
    gh[NSTEP - 4].wait()
    gh[NSTEP - 3].wait()
    gh[NSTEP - 2].wait()
    sh[NBATCH - 1] = writeback(NBATCH - 1, 3)
    gh[NSTEP - 1].wait()
    sh[NBATCH] = writeback(NBATCH, 1)
    sh[NBATCH - 1].wait()
    sh[NBATCH].wait()

  return k(table, idx)


def kernel(inputs, embeddings):
  return _sc_gather(embeddings.astype(jnp.float32), inputs.astype(jnp.int32))
